# baseline (speedup 1.0000x reference)
import functools

import jax
import jax.numpy as jnp
from jax import lax
from jax.experimental import pallas as pl
from jax.experimental.pallas import tpu as pltpu
from jax.experimental.pallas import tpu_sc as plsc

_NC = 2
_NS = 16
_NW = _NC * _NS
_LANES = 16
_CHUNK = 128

_N = 10000
_NPAD = 10240
_RPT = _NPAD // _NS
_FIN = 128
_HID = 128
_CLS = 40
_CLSP = 48
_BLK = 512
_NBUF = 4


def _mesh():
    return plsc.VectorSubcoreMesh(
        core_axis_name="c", subcore_axis_name="s",
        num_cores=_NC, num_subcores=_NS)


_SC_PARAMS = pltpu.CompilerParams(use_tc_tiling_on_sc=False)


def _edge_pipeline(tab_hbm, acc, sidx, didx, rows, gsems, ssems, nch,
                   nbuf=_NBUF):
    depth = nbuf // 2
    for b in range(depth):
        pltpu.async_copy(tab_hbm.at[sidx.at[b]], rows.at[b], gsems[b])

    @pl.loop(0, nch, step=nbuf)
    def _(j):
        for b in range(nbuf):
            c = j + b
            b2 = (b + depth) % nbuf
            pltpu.make_async_copy(
                tab_hbm.at[sidx.at[c]], rows.at[b], gsems[b]).wait()
            pltpu.async_copy(rows.at[b], acc.at[didx.at[c]], ssems[b],
                             add=True)

            @pl.when(c + depth < nch)
            def _():
                @pl.when(c >= nbuf - depth)
                def _():
                    pltpu.make_async_copy(
                        rows.at[b2], acc.at[didx.at[c - (nbuf - depth)]],
                        ssems[b2]).wait()
                pltpu.async_copy(
                    tab_hbm.at[sidx.at[c + depth]], rows.at[b2], gsems[b2])

    for b in range(nbuf):
        pltpu.make_async_copy(
            rows.at[b], acc.at[didx.at[nch - nbuf + b]], ssems[b]).wait()


def _sc_degree(dst2d, ch):
    @functools.partial(
        pl.kernel,
        out_type=jax.ShapeDtypeStruct((_NC, _NPAD, _LANES), jnp.float32),
        mesh=_mesh(),
        compiler_params=_SC_PARAMS,
        scratch_types=[
            pltpu.VMEM((ch, _CHUNK), jnp.int32),
            pltpu.VMEM((_CHUNK, _LANES), jnp.float32),
            pltpu.VMEM((_RPT, _LANES), jnp.float32),
            pltpu.VMEM_SHARED((_NPAD, _LANES), jnp.float32),
        ],
    )
    def k(dst_hbm, out_hbm, idx_v, ones_v, zero_v, acc):
        cid = lax.axis_index("c")
        sid = lax.axis_index("s")
        wid = sid * _NC + cid

        @pl.loop(0, _CHUNK)
        def _(r):
            ones_v[r] = jnp.ones((_LANES,), jnp.float32)

        @pl.loop(0, _RPT)
        def _(r):
            zero_v[r] = jnp.zeros((_LANES,), jnp.float32)

        row0 = sid * _RPT
        pltpu.sync_copy(zero_v, acc.at[pl.ds(row0, _RPT)])
        pltpu.sync_copy(dst_hbm.at[pl.ds(wid * ch, ch)], idx_v)
        plsc.subcore_barrier()

        @pl.loop(0, ch)
        def _(j):
            pltpu.sync_copy(ones_v, acc.at[idx_v.at[j]], add=True)

        plsc.subcore_barrier()
        pltpu.sync_copy(acc.at[pl.ds(row0, _RPT)],
                        out_hbm.at[cid, pl.ds(row0, _RPT)])

    return k(dst2d)


def _sc_propagate_cols(table2, src2d, dst2d, ch2):
    hw = _HID // _NC

    @functools.partial(
        pl.kernel,
        out_type=jax.ShapeDtypeStruct((_NPAD, _HID), jnp.float32),
        mesh=_mesh(),
        compiler_params=_SC_PARAMS,
        scratch_types=[
            pltpu.VMEM((ch2, _CHUNK), jnp.int32),
            pltpu.VMEM((ch2, _CHUNK), jnp.int32),
            pltpu.VMEM((_NBUF, _CHUNK, hw), jnp.float32),
            pltpu.VMEM_SHARED((_NPAD, hw), jnp.float32),
            [pltpu.SemaphoreType.DMA] * _NBUF,
            [pltpu.SemaphoreType.DMA] * _NBUF,
        ],
    )
    def k(tab_hbm, src_hbm, dst_hbm, out_hbm, sidx, didx, rows, acc,
          gsems, ssems):
        cid = lax.axis_index("c")
        sid = lax.axis_index("s")

        @pl.loop(0, _CHUNK)
        def _(r):
            @pl.loop(0, hw // _LANES)
            def _(q):
                rows[0, r, pl.ds(q * _LANES, _LANES)] = jnp.zeros(
                    (_LANES,), jnp.float32)

        row0 = sid * _RPT

        @pl.loop(0, _RPT // _CHUNK)
        def _(i):
            pltpu.sync_copy(rows.at[0],
                            acc.at[pl.ds(row0 + i * _CHUNK, _CHUNK)])

        pltpu.sync_copy(src_hbm.at[pl.ds(sid * ch2, ch2)], sidx)
        pltpu.sync_copy(dst_hbm.at[pl.ds(sid * ch2, ch2)], didx)
        plsc.subcore_barrier()
        _edge_pipeline(tab_hbm.at[cid], acc, sidx, didx, rows, gsems, ssems,
                       ch2)
        plsc.subcore_barrier()
        pltpu.sync_copy(acc.at[pl.ds(row0, _RPT)],
                        out_hbm.at[pl.ds(row0, _RPT), pl.ds(cid * hw, hw)])

    return k(table2, src2d, dst2d)


def _sc_propagate(table, src2d, dst2d, ch0, ch1, width, nbuf=4):
    chmax = max(ch0, ch1)
    assert ch0 % nbuf == 0 and ch1 % nbuf == 0

    @functools.partial(
        pl.kernel,
        out_type=jax.ShapeDtypeStruct((_NC, _NPAD, width), jnp.float32),
        mesh=_mesh(),
        compiler_params=_SC_PARAMS,
        scratch_types=[
            pltpu.VMEM((chmax, _CHUNK), jnp.int32),
            pltpu.VMEM((chmax, _CHUNK), jnp.int32),
            pltpu.VMEM((nbuf, _CHUNK, width), jnp.float32),
            pltpu.VMEM_SHARED((_NPAD, width), jnp.float32),
            [pltpu.SemaphoreType.DMA] * nbuf,
            [pltpu.SemaphoreType.DMA] * nbuf,
        ],
    )
    def k(tab_hbm, src_hbm, dst_hbm, out_hbm, sidx, didx, rows, acc,
          gsems, ssems):
        cid = lax.axis_index("c")
        sid = lax.axis_index("s")

        @pl.loop(0, _CHUNK)
        def _(r):
            @pl.loop(0, width // _LANES)
            def _(q):
                rows[0, r, pl.ds(q * _LANES, _LANES)] = jnp.zeros(
                    (_LANES,), jnp.float32)

        row0 = sid * _RPT

        @pl.loop(0, _RPT // _CHUNK)
        def _(i):
            pltpu.sync_copy(rows.at[0],
                            acc.at[pl.ds(row0 + i * _CHUNK, _CHUNK)])

        if ch0 > 0:
            @pl.when(cid == 0)
            def _():
                pltpu.sync_copy(src_hbm.at[pl.ds(sid * ch0, ch0)],
                                sidx.at[pl.ds(0, ch0)])
                pltpu.sync_copy(dst_hbm.at[pl.ds(sid * ch0, ch0)],
                                didx.at[pl.ds(0, ch0)])
                plsc.subcore_barrier()
                _edge_pipeline(tab_hbm, acc, sidx, didx, rows,
                               gsems, ssems, ch0, nbuf)

        if ch1 > 0:
            @pl.when(cid == 1)
            def _():
                pltpu.sync_copy(
                    src_hbm.at[pl.ds(_NS * ch0 + sid * ch1, ch1)],
                    sidx.at[pl.ds(0, ch1)])
                pltpu.sync_copy(
                    dst_hbm.at[pl.ds(_NS * ch0 + sid * ch1, ch1)],
                    didx.at[pl.ds(0, ch1)])
                plsc.subcore_barrier()
                _edge_pipeline(tab_hbm, acc, sidx, didx, rows,
                               gsems, ssems, ch1, nbuf)

        plsc.subcore_barrier()
        pltpu.sync_copy(acc.at[pl.ds(row0, _RPT)],
                        out_hbm.at[cid, pl.ds(row0, _RPT)])

    return k(table, src2d, dst2d)


def _dd(deg_ref):
    deg = deg_ref[0, :, 0:1] + deg_ref[1, :, 0:1] + 1.0
    return lax.rsqrt(deg)


def _tc_mm1(xpad, w1s):
    hw = _HID // _NC
    nrb = _NPAD // _BLK

    def body(x_ref, w_ref, o_ref):
        o_ref[0] = jnp.dot(x_ref[...], w_ref[0],
                           preferred_element_type=jnp.float32)

    return pl.pallas_call(
        body,
        grid=(nrb, _NC),
        in_specs=[
            pl.BlockSpec((_BLK, _FIN), lambda i, j: (i, 0)),
            pl.BlockSpec((1, _FIN, hw), lambda i, j: (j, 0, 0)),
        ],
        out_specs=pl.BlockSpec((1, _BLK, hw), lambda i, j: (j, i, 0)),
        out_shape=jax.ShapeDtypeStruct((_NC, _NPAD, hw), jnp.float32),
    )(xpad, w1s)


def _tc1(xpad, w1s, degp):
    hw = _HID // _NC
    nrb = _NPAD // _BLK

    def body(x_ref, w_ref, deg_ref, o_ref):
        xw = jnp.dot(x_ref[...], w_ref[0],
                     preferred_element_type=jnp.float32)
        o_ref[0] = xw * _dd(deg_ref)

    return pl.pallas_call(
        body,
        grid=(nrb, _NC),
        in_specs=[
            pl.BlockSpec((_BLK, _FIN), lambda i, j: (i, 0)),
            pl.BlockSpec((1, _FIN, hw), lambda i, j: (j, 0, 0)),
            pl.BlockSpec((_NC, _BLK, _LANES), lambda i, j: (0, i, 0)),
        ],
        out_specs=pl.BlockSpec((1, _BLK, hw), lambda i, j: (j, i, 0)),
        out_shape=jax.ShapeDtypeStruct((_NC, _NPAD, hw), jnp.float32),
    )(xpad, w1s, degp)


def _tc2(pp, table2, degp, w2p, b1r):
    hw = _HID // _NC
    nrb = _NPAD // _BLK

    def body(p_ref, xsl_ref, xsr_ref, deg_ref, w_ref, b_ref, o_ref):
        dd = _dd(deg_ref)
        xs = jnp.concatenate([xsl_ref[0], xsr_ref[0]], axis=1)
        h = jnp.maximum(dd * (p_ref[...] + xs) + b_ref[...], 0.0)
        o_ref[...] = jnp.dot(h, w_ref[...],
                             preferred_element_type=jnp.float32) * dd

    return pl.pallas_call(
        body,
        grid=(nrb,),
        in_specs=[
            pl.BlockSpec((_BLK, _HID), lambda i: (i, 0)),
            pl.BlockSpec((1, _BLK, hw), lambda i: (0, i, 0)),
            pl.BlockSpec((1, _BLK, hw), lambda i: (1, i, 0)),
            pl.BlockSpec((_NC, _BLK, _LANES), lambda i: (0, i, 0)),
            pl.BlockSpec((_HID, _CLSP), lambda i: (0, 0)),
            pl.BlockSpec((1, _HID), lambda i: (0, 0)),
        ],
        out_specs=pl.BlockSpec((_BLK, _CLSP), lambda i: (i, 0)),
        out_shape=jax.ShapeDtypeStruct((_NPAD, _CLSP), jnp.float32),
    )(pp, table2, table2, degp, w2p, b1r)


def _tc3(qp, hw_s, degp, b2r):
    def body(q_ref, hs_ref, deg_ref, b_ref, o_ref):
        logits = (_dd(deg_ref) * (q_ref[0] + q_ref[1] + hs_ref[...])
                  + b_ref[...])
        col = lax.broadcasted_iota(jnp.int32, (_BLK, _CLSP), 1)
        logits = jnp.where(col < _CLS, logits, -1e30)
        m = jnp.max(logits, axis=1, keepdims=True)
        lse = jnp.log(jnp.sum(jnp.exp(logits - m), axis=1, keepdims=True))
        o_ref[...] = logits - m - lse

    return pl.pallas_call(
        body,
        grid=(_NPAD // _BLK,),
        in_specs=[
            pl.BlockSpec((_NC, _BLK, _CLSP), lambda i: (0, i, 0)),
            pl.BlockSpec((_BLK, _CLSP), lambda i: (i, 0)),
            pl.BlockSpec((_NC, _BLK, _LANES), lambda i: (0, i, 0)),
            pl.BlockSpec((1, _CLSP), lambda i: (0, 0)),
        ],
        out_specs=pl.BlockSpec((_BLK, _CLSP), lambda i: (i, 0)),
        out_shape=jax.ShapeDtypeStruct((_NPAD, _CLSP), jnp.float32),
    )(qp, hw_s, degp, b2r)


def kernel(x, edge_index, W1, b1, W2, b2):
    e = edge_index.shape[1]
    ei = edge_index.astype(jnp.int32)

    per_round = _NBUF * _NW * _CHUNK
    epad = -(-e // per_round) * per_round
    ch = epad // (_NW * _CHUNK)
    ch2 = _NC * ch
    src = jnp.pad(ei[0], (0, epad - e), constant_values=_N)
    dst = jnp.pad(ei[1], (0, epad - e), constant_values=_N)
    src2d = src.reshape(_NS * ch2, _CHUNK)
    dst2d = dst.reshape(_NS * ch2, _CHUNK)

    xpad = jnp.pad(x, ((0, _NPAD - _N), (0, 0)))
    w1s = W1.reshape(_FIN, _NC, _HID // _NC).transpose(1, 0, 2)
    w2p = jnp.pad(W2, ((0, 0), (0, _CLSP - _CLS)))
    b1r = b1.reshape(1, _HID)
    b2r = jnp.pad(b2, (0, _CLSP - _CLS)).reshape(1, _CLSP)

    ch1 = max(_NBUF, (2 * ch) // 4 // _NBUF * _NBUF)
    ch0 = 2 * ch - ch1

    degp = _sc_degree(dst2d, ch)
    table2 = _tc1(xpad, w1s, degp)
    pp = _sc_propagate_cols(table2, src2d, dst2d, ch2)
    hw_s = _tc2(pp, table2, degp, w2p, b1r)
    qp = _sc_propagate(hw_s, src2d, dst2d, ch0, ch1, _CLSP)
    out = _tc3(qp, hw_s, degp, b2r)
    return out[:_N, :_CLS]


# speedup vs baseline: 1.1043x; 1.1043x over previous
import functools

import jax
import jax.numpy as jnp
from jax import lax
from jax.experimental import pallas as pl
from jax.experimental.pallas import tpu as pltpu
from jax.experimental.pallas import tpu_sc as plsc

_NC = 2
_NS = 16
_NW = _NC * _NS
_LANES = 16
_CHUNK = 128

_N = 10000
_NPAD = 10240
_RPT = _NPAD // _NS
_FIN = 128
_HID = 128
_CLS = 40
_CLSP = 48
_BLK = 512
_NBUF = 4


def _mesh():
    return plsc.VectorSubcoreMesh(
        core_axis_name="c", subcore_axis_name="s",
        num_cores=_NC, num_subcores=_NS)


_SC_PARAMS = pltpu.CompilerParams(use_tc_tiling_on_sc=False)


def _edge_pipeline(tab_hbm, acc, sidx, didx, rows, gsems, ssems, nch,
                   nbuf=_NBUF):
    depth = nbuf // 2
    for b in range(depth):
        pltpu.async_copy(tab_hbm.at[sidx.at[b]], rows.at[b], gsems[b])

    @pl.loop(0, nch, step=nbuf)
    def _(j):
        for b in range(nbuf):
            c = j + b
            b2 = (b + depth) % nbuf
            pltpu.make_async_copy(
                tab_hbm.at[sidx.at[c]], rows.at[b], gsems[b]).wait()
            pltpu.async_copy(rows.at[b], acc.at[didx.at[c]], ssems[b],
                             add=True)

            @pl.when(c + depth < nch)
            def _():
                @pl.when(c >= nbuf - depth)
                def _():
                    pltpu.make_async_copy(
                        rows.at[b2], acc.at[didx.at[c - (nbuf - depth)]],
                        ssems[b2]).wait()
                pltpu.async_copy(
                    tab_hbm.at[sidx.at[c + depth]], rows.at[b2], gsems[b2])

    for b in range(nbuf):
        pltpu.make_async_copy(
            rows.at[b], acc.at[didx.at[nch - nbuf + b]], ssems[b]).wait()


def _sc_degree(dst2d, ch):
    @functools.partial(
        pl.kernel,
        out_type=jax.ShapeDtypeStruct((_NC, _NPAD, _LANES), jnp.float32),
        mesh=_mesh(),
        compiler_params=_SC_PARAMS,
        scratch_types=[
            pltpu.VMEM((ch, _CHUNK), jnp.int32),
            pltpu.VMEM((_CHUNK, _LANES), jnp.float32),
            pltpu.VMEM((_RPT, _LANES), jnp.float32),
            pltpu.VMEM_SHARED((_NPAD, _LANES), jnp.float32),
        ],
    )
    def k(dst_hbm, out_hbm, idx_v, ones_v, zero_v, acc):
        cid = lax.axis_index("c")
        sid = lax.axis_index("s")
        wid = sid * _NC + cid

        @pl.loop(0, _CHUNK)
        def _(r):
            ones_v[r] = jnp.ones((_LANES,), jnp.float32)

        @pl.loop(0, _RPT)
        def _(r):
            zero_v[r] = jnp.zeros((_LANES,), jnp.float32)

        row0 = sid * _RPT
        pltpu.sync_copy(zero_v, acc.at[pl.ds(row0, _RPT)])
        pltpu.sync_copy(dst_hbm.at[pl.ds(wid * ch, ch)], idx_v)
        plsc.subcore_barrier()

        @pl.loop(0, ch)
        def _(j):
            pltpu.sync_copy(ones_v, acc.at[idx_v.at[j]], add=True)

        plsc.subcore_barrier()
        pltpu.sync_copy(acc.at[pl.ds(row0, _RPT)],
                        out_hbm.at[cid, pl.ds(row0, _RPT)])

    return k(dst2d)


def _sc_propagate_cols(table2, srcoff2d, dst2d, ch2):
    hw = _HID // _NC

    @functools.partial(
        pl.kernel,
        out_type=jax.ShapeDtypeStruct((_NPAD, _HID), jnp.float32),
        mesh=_mesh(),
        compiler_params=_SC_PARAMS,
        scratch_types=[
            pltpu.VMEM((ch2, _CHUNK), jnp.int32),
            pltpu.VMEM((ch2, _CHUNK), jnp.int32),
            pltpu.VMEM((_NBUF, _CHUNK, hw), jnp.float32),
            pltpu.VMEM_SHARED((_NPAD, hw), jnp.float32),
            [pltpu.SemaphoreType.DMA] * _NBUF,
            [pltpu.SemaphoreType.DMA] * _NBUF,
        ],
    )
    def k(tab_hbm, src_hbm, dst_hbm, out_hbm, sidx, didx, rows, acc,
          gsems, ssems):
        cid = lax.axis_index("c")
        sid = lax.axis_index("s")

        @pl.loop(0, _CHUNK)
        def _(r):
            @pl.loop(0, hw // _LANES)
            def _(q):
                rows[0, r, pl.ds(q * _LANES, _LANES)] = jnp.zeros(
                    (_LANES,), jnp.float32)

        row0 = sid * _RPT

        @pl.loop(0, _RPT // _CHUNK)
        def _(i):
            pltpu.sync_copy(rows.at[0],
                            acc.at[pl.ds(row0 + i * _CHUNK, _CHUNK)])

        pltpu.sync_copy(src_hbm.at[pl.ds((cid * _NS + sid) * ch2, ch2)], sidx)
        pltpu.sync_copy(dst_hbm.at[pl.ds(sid * ch2, ch2)], didx)
        plsc.subcore_barrier()
        _edge_pipeline(tab_hbm, acc, sidx, didx, rows, gsems, ssems,
                       ch2)
        plsc.subcore_barrier()
        pltpu.sync_copy(acc.at[pl.ds(row0, _RPT)],
                        out_hbm.at[pl.ds(row0, _RPT), pl.ds(cid * hw, hw)])

    return k(table2, srcoff2d, dst2d)


def _sc_propagate(table, src2d, dst2d, ch0, ch1, width, nbuf=4):
    chmax = max(ch0, ch1)
    assert ch0 % nbuf == 0 and ch1 % nbuf == 0

    @functools.partial(
        pl.kernel,
        out_type=jax.ShapeDtypeStruct((_NC, _NPAD, width), jnp.float32),
        mesh=_mesh(),
        compiler_params=_SC_PARAMS,
        scratch_types=[
            pltpu.VMEM((chmax, _CHUNK), jnp.int32),
            pltpu.VMEM((chmax, _CHUNK), jnp.int32),
            pltpu.VMEM((nbuf, _CHUNK, width), jnp.float32),
            pltpu.VMEM_SHARED((_NPAD, width), jnp.float32),
            [pltpu.SemaphoreType.DMA] * nbuf,
            [pltpu.SemaphoreType.DMA] * nbuf,
        ],
    )
    def k(tab_hbm, src_hbm, dst_hbm, out_hbm, sidx, didx, rows, acc,
          gsems, ssems):
        cid = lax.axis_index("c")
        sid = lax.axis_index("s")

        @pl.loop(0, _CHUNK)
        def _(r):
            @pl.loop(0, width // _LANES)
            def _(q):
                rows[0, r, pl.ds(q * _LANES, _LANES)] = jnp.zeros(
                    (_LANES,), jnp.float32)

        row0 = sid * _RPT

        @pl.loop(0, _RPT // _CHUNK)
        def _(i):
            pltpu.sync_copy(rows.at[0],
                            acc.at[pl.ds(row0 + i * _CHUNK, _CHUNK)])

        if ch0 > 0:
            @pl.when(cid == 0)
            def _():
                pltpu.sync_copy(src_hbm.at[pl.ds(sid * ch0, ch0)],
                                sidx.at[pl.ds(0, ch0)])
                pltpu.sync_copy(dst_hbm.at[pl.ds(sid * ch0, ch0)],
                                didx.at[pl.ds(0, ch0)])
                plsc.subcore_barrier()
                _edge_pipeline(tab_hbm, acc, sidx, didx, rows,
                               gsems, ssems, ch0, nbuf)

        if ch1 > 0:
            @pl.when(cid == 1)
            def _():
                pltpu.sync_copy(
                    src_hbm.at[pl.ds(_NS * ch0 + sid * ch1, ch1)],
                    sidx.at[pl.ds(0, ch1)])
                pltpu.sync_copy(
                    dst_hbm.at[pl.ds(_NS * ch0 + sid * ch1, ch1)],
                    didx.at[pl.ds(0, ch1)])
                plsc.subcore_barrier()
                _edge_pipeline(tab_hbm, acc, sidx, didx, rows,
                               gsems, ssems, ch1, nbuf)

        plsc.subcore_barrier()
        pltpu.sync_copy(acc.at[pl.ds(row0, _RPT)],
                        out_hbm.at[cid, pl.ds(row0, _RPT)])

    return k(table, src2d, dst2d)


def _dd(deg_ref):
    deg = deg_ref[0, :, 0:1] + deg_ref[1, :, 0:1] + 1.0
    return lax.rsqrt(deg)


def _tc_mm1(xpad, w1s):
    hw = _HID // _NC
    nrb = _NPAD // _BLK

    def body(x_ref, w_ref, o_ref):
        o_ref[...] = jnp.dot(x_ref[...], w_ref[0],
                             preferred_element_type=jnp.float32)

    return pl.pallas_call(
        body,
        grid=(nrb, _NC),
        in_specs=[
            pl.BlockSpec((_BLK, _FIN), lambda i, j: (i, 0)),
            pl.BlockSpec((1, _FIN, hw), lambda i, j: (j, 0, 0)),
        ],
        out_specs=pl.BlockSpec((_BLK, hw), lambda i, j: (j * nrb + i, 0)),
        out_shape=jax.ShapeDtypeStruct((_NC * _NPAD, hw), jnp.float32),
    )(xpad, w1s)


def _tc_scale1(xw2, degp):
    hw = _HID // _NC
    nrb = _NPAD // _BLK

    def body(xw_ref, deg_ref, o_ref):
        o_ref[...] = xw_ref[...] * _dd(deg_ref)

    return pl.pallas_call(
        body,
        grid=(nrb, _NC),
        in_specs=[
            pl.BlockSpec((_BLK, hw), lambda i, j: (j * nrb + i, 0)),
            pl.BlockSpec((_NC, _BLK, _LANES), lambda i, j: (0, i, 0)),
        ],
        out_specs=pl.BlockSpec((_BLK, hw), lambda i, j: (j * nrb + i, 0)),
        out_shape=jax.ShapeDtypeStruct((_NC * _NPAD, hw), jnp.float32),
    )(xw2, degp)


def _tc2(pp, table2, degp, w2p, b1r):
    hw = _HID // _NC
    nrb = _NPAD // _BLK

    def body(p_ref, xsl_ref, xsr_ref, deg_ref, w_ref, b_ref, o_ref):
        dd = _dd(deg_ref)
        xs = jnp.concatenate([xsl_ref[...], xsr_ref[...]], axis=1)
        h = jnp.maximum(dd * (p_ref[...] + xs) + b_ref[...], 0.0)
        o_ref[...] = jnp.dot(h, w_ref[...],
                             preferred_element_type=jnp.float32) * dd

    return pl.pallas_call(
        body,
        grid=(nrb,),
        in_specs=[
            pl.BlockSpec((_BLK, _HID), lambda i: (i, 0)),
            pl.BlockSpec((_BLK, hw), lambda i: (i, 0)),
            pl.BlockSpec((_BLK, hw), lambda i: (nrb + i, 0)),
            pl.BlockSpec((_NC, _BLK, _LANES), lambda i: (0, i, 0)),
            pl.BlockSpec((_HID, _CLSP), lambda i: (0, 0)),
            pl.BlockSpec((1, _HID), lambda i: (0, 0)),
        ],
        out_specs=pl.BlockSpec((_BLK, _CLSP), lambda i: (i, 0)),
        out_shape=jax.ShapeDtypeStruct((_NPAD, _CLSP), jnp.float32),
    )(pp, table2, table2, degp, w2p, b1r)


def _tc3(qp, hw_s, degp, b2r):
    def body(q_ref, hs_ref, deg_ref, b_ref, o_ref):
        logits = (_dd(deg_ref) * (q_ref[0] + q_ref[1] + hs_ref[...])
                  + b_ref[...])
        col = lax.broadcasted_iota(jnp.int32, (_BLK, _CLSP), 1)
        logits = jnp.where(col < _CLS, logits, -1e30)
        m = jnp.max(logits, axis=1, keepdims=True)
        lse = jnp.log(jnp.sum(jnp.exp(logits - m), axis=1, keepdims=True))
        o_ref[...] = logits - m - lse

    return pl.pallas_call(
        body,
        grid=(_NPAD // _BLK,),
        in_specs=[
            pl.BlockSpec((_NC, _BLK, _CLSP), lambda i: (0, i, 0)),
            pl.BlockSpec((_BLK, _CLSP), lambda i: (i, 0)),
            pl.BlockSpec((_NC, _BLK, _LANES), lambda i: (0, i, 0)),
            pl.BlockSpec((1, _CLSP), lambda i: (0, 0)),
        ],
        out_specs=pl.BlockSpec((_BLK, _CLSP), lambda i: (i, 0)),
        out_shape=jax.ShapeDtypeStruct((_NPAD, _CLSP), jnp.float32),
    )(qp, hw_s, degp, b2r)


def kernel(x, edge_index, W1, b1, W2, b2):
    e = edge_index.shape[1]
    ei = edge_index.astype(jnp.int32)

    per_round = _NBUF * _NW * _CHUNK
    epad = -(-e // per_round) * per_round
    ch = epad // (_NW * _CHUNK)
    ch2 = _NC * ch
    src = jnp.pad(ei[0], (0, epad - e), constant_values=_N)
    dst = jnp.pad(ei[1], (0, epad - e), constant_values=_N)
    src2d = src.reshape(_NS * ch2, _CHUNK)
    dst2d = dst.reshape(_NS * ch2, _CHUNK)
    srcoff2d = jnp.concatenate([src2d, src2d + _NPAD], axis=0)

    xpad = jnp.pad(x, ((0, _NPAD - _N), (0, 0)))
    w1s = W1.reshape(_FIN, _NC, _HID // _NC).transpose(1, 0, 2)
    w2p = jnp.pad(W2, ((0, 0), (0, _CLSP - _CLS)))
    b1r = b1.reshape(1, _HID)
    b2r = jnp.pad(b2, (0, _CLSP - _CLS)).reshape(1, _CLSP)

    ch1 = max(_NBUF, (2 * ch) // 4 // _NBUF * _NBUF)
    ch0 = 2 * ch - ch1

    xw2 = _tc_mm1(xpad, w1s)
    degp = _sc_degree(dst2d, ch)
    table2 = _tc_scale1(xw2, degp)
    pp = _sc_propagate_cols(table2, srcoff2d, dst2d, ch2)
    hw_s = _tc2(pp, table2, degp, w2p, b1r)
    qp = _sc_propagate(hw_s, src2d, dst2d, ch0, ch1, _CLSP)
    out = _tc3(qp, hw_s, degp, b2r)
    return out[:_N, :_CLS]


# speedup vs baseline: 1.1115x; 1.0065x over previous
import functools

import jax
import jax.numpy as jnp
from jax import lax
from jax.experimental import pallas as pl
from jax.experimental.pallas import tpu as pltpu
from jax.experimental.pallas import tpu_sc as plsc

_NC = 2
_NS = 16
_NW = _NC * _NS
_LANES = 16
_CHUNK = 128

_N = 10000
_NPAD = 10240
_RPT = _NPAD // _NS
_FIN = 128
_HID = 128
_CLS = 40
_CLSP = 48
_BLK = 512
_NBUF = 4


def _mesh():
    return plsc.VectorSubcoreMesh(
        core_axis_name="c", subcore_axis_name="s",
        num_cores=_NC, num_subcores=_NS)


_SC_PARAMS = pltpu.CompilerParams(use_tc_tiling_on_sc=False)


def _edge_pipeline(tab_hbm, acc, sidx, didx, rows, gsems, ssems, nch,
                   nbuf=_NBUF):
    depth = nbuf // 2
    for b in range(depth):
        pltpu.async_copy(tab_hbm.at[sidx.at[b]], rows.at[b], gsems[b])

    @pl.loop(0, nch, step=nbuf)
    def _(j):
        for b in range(nbuf):
            c = j + b
            b2 = (b + depth) % nbuf
            pltpu.make_async_copy(
                tab_hbm.at[sidx.at[c]], rows.at[b], gsems[b]).wait()
            pltpu.async_copy(rows.at[b], acc.at[didx.at[c]], ssems[b],
                             add=True)

            @pl.when(c + depth < nch)
            def _():
                @pl.when(c >= nbuf - depth)
                def _():
                    pltpu.make_async_copy(
                        rows.at[b2], acc.at[didx.at[c - (nbuf - depth)]],
                        ssems[b2]).wait()
                pltpu.async_copy(
                    tab_hbm.at[sidx.at[c + depth]], rows.at[b2], gsems[b2])

    for b in range(nbuf):
        pltpu.make_async_copy(
            rows.at[b], acc.at[didx.at[nch - nbuf + b]], ssems[b]).wait()


def _sc_degree(dst2d, ch):
    @functools.partial(
        pl.kernel,
        out_type=jax.ShapeDtypeStruct((_NC, _NPAD, _LANES), jnp.float32),
        mesh=_mesh(),
        compiler_params=_SC_PARAMS,
        scratch_types=[
            pltpu.VMEM((ch, _CHUNK), jnp.int32),
            pltpu.VMEM((_CHUNK, _LANES), jnp.float32),
            pltpu.VMEM((_RPT, _LANES), jnp.float32),
            pltpu.VMEM_SHARED((_NPAD, _LANES), jnp.float32),
        ],
    )
    def k(dst_hbm, out_hbm, idx_v, ones_v, zero_v, acc):
        cid = lax.axis_index("c")
        sid = lax.axis_index("s")
        wid = sid * _NC + cid

        @pl.loop(0, _CHUNK)
        def _(r):
            ones_v[r] = jnp.ones((_LANES,), jnp.float32)

        @pl.loop(0, _RPT)
        def _(r):
            zero_v[r] = jnp.zeros((_LANES,), jnp.float32)

        row0 = sid * _RPT
        pltpu.sync_copy(zero_v, acc.at[pl.ds(row0, _RPT)])
        pltpu.sync_copy(dst_hbm.at[pl.ds(wid * ch, ch)], idx_v)
        plsc.subcore_barrier()

        @pl.loop(0, ch)
        def _(j):
            pltpu.sync_copy(ones_v, acc.at[idx_v.at[j]], add=True)

        plsc.subcore_barrier()
        pltpu.sync_copy(acc.at[pl.ds(row0, _RPT)],
                        out_hbm.at[cid, pl.ds(row0, _RPT)])

    return k(dst2d)


def _sc_propagate_cols(table2, srcoff2d, dst2d, ch2):
    hw = _HID // _NC

    @functools.partial(
        pl.kernel,
        out_type=jax.ShapeDtypeStruct((_NPAD, _HID), jnp.float32),
        mesh=_mesh(),
        compiler_params=_SC_PARAMS,
        scratch_types=[
            pltpu.VMEM((ch2, _CHUNK), jnp.int32),
            pltpu.VMEM((ch2, _CHUNK), jnp.int32),
            pltpu.VMEM((_NBUF, _CHUNK, hw), jnp.float32),
            pltpu.VMEM_SHARED((_NPAD, hw), jnp.float32),
            [pltpu.SemaphoreType.DMA] * _NBUF,
            [pltpu.SemaphoreType.DMA] * _NBUF,
        ],
    )
    def k(tab_hbm, src_hbm, dst_hbm, out_hbm, sidx, didx, rows, acc,
          gsems, ssems):
        cid = lax.axis_index("c")
        sid = lax.axis_index("s")

        @pl.loop(0, _CHUNK)
        def _(r):
            @pl.loop(0, hw // _LANES)
            def _(q):
                rows[0, r, pl.ds(q * _LANES, _LANES)] = jnp.zeros(
                    (_LANES,), jnp.float32)

        row0 = sid * _RPT

        @pl.loop(0, _RPT // _CHUNK)
        def _(i):
            pltpu.sync_copy(rows.at[0],
                            acc.at[pl.ds(row0 + i * _CHUNK, _CHUNK)])

        pltpu.sync_copy(src_hbm.at[pl.ds((cid * _NS + sid) * ch2, ch2)], sidx)
        pltpu.sync_copy(dst_hbm.at[pl.ds(sid * ch2, ch2)], didx)
        plsc.subcore_barrier()
        _edge_pipeline(tab_hbm, acc, sidx, didx, rows, gsems, ssems,
                       ch2)
        plsc.subcore_barrier()
        pltpu.sync_copy(acc.at[pl.ds(row0, _RPT)],
                        out_hbm.at[pl.ds(row0, _RPT), pl.ds(cid * hw, hw)])

    return k(table2, srcoff2d, dst2d)


def _sc_propagate(table, src2d, dst2d, ch0, ch1, width, nbuf=4):
    chmax = max(ch0, ch1)
    assert ch0 % nbuf == 0 and ch1 % nbuf == 0

    @functools.partial(
        pl.kernel,
        out_type=jax.ShapeDtypeStruct((_NC, _NPAD, width), jnp.float32),
        mesh=_mesh(),
        compiler_params=_SC_PARAMS,
        scratch_types=[
            pltpu.VMEM((chmax, _CHUNK), jnp.int32),
            pltpu.VMEM((chmax, _CHUNK), jnp.int32),
            pltpu.VMEM((nbuf, _CHUNK, width), jnp.float32),
            pltpu.VMEM_SHARED((_NPAD, width), jnp.float32),
            [pltpu.SemaphoreType.DMA] * nbuf,
            [pltpu.SemaphoreType.DMA] * nbuf,
        ],
    )
    def k(tab_hbm, src_hbm, dst_hbm, out_hbm, sidx, didx, rows, acc,
          gsems, ssems):
        cid = lax.axis_index("c")
        sid = lax.axis_index("s")

        @pl.loop(0, _CHUNK)
        def _(r):
            @pl.loop(0, width // _LANES)
            def _(q):
                rows[0, r, pl.ds(q * _LANES, _LANES)] = jnp.zeros(
                    (_LANES,), jnp.float32)

        row0 = sid * _RPT

        @pl.loop(0, _RPT // _CHUNK)
        def _(i):
            pltpu.sync_copy(rows.at[0],
                            acc.at[pl.ds(row0 + i * _CHUNK, _CHUNK)])

        if ch0 > 0:
            @pl.when(cid == 0)
            def _():
                pltpu.sync_copy(src_hbm.at[pl.ds(sid * ch0, ch0)],
                                sidx.at[pl.ds(0, ch0)])
                pltpu.sync_copy(dst_hbm.at[pl.ds(sid * ch0, ch0)],
                                didx.at[pl.ds(0, ch0)])
                plsc.subcore_barrier()
                _edge_pipeline(tab_hbm, acc, sidx, didx, rows,
                               gsems, ssems, ch0, nbuf)

        if ch1 > 0:
            @pl.when(cid == 1)
            def _():
                pltpu.sync_copy(
                    src_hbm.at[pl.ds(_NS * ch0 + sid * ch1, ch1)],
                    sidx.at[pl.ds(0, ch1)])
                pltpu.sync_copy(
                    dst_hbm.at[pl.ds(_NS * ch0 + sid * ch1, ch1)],
                    didx.at[pl.ds(0, ch1)])
                plsc.subcore_barrier()
                _edge_pipeline(tab_hbm, acc, sidx, didx, rows,
                               gsems, ssems, ch1, nbuf)

        plsc.subcore_barrier()
        pltpu.sync_copy(acc.at[pl.ds(row0, _RPT)],
                        out_hbm.at[cid, pl.ds(row0, _RPT)])

    return k(table, src2d, dst2d)


def _dd(deg_ref):
    deg = deg_ref[0, :, 0:1] + deg_ref[1, :, 0:1] + 1.0
    return lax.rsqrt(deg)


def _tc_mm1(xpad, w1s):
    hw = _HID // _NC
    nrb = _NPAD // _BLK

    def body(x_ref, w_ref, o_ref):
        o_ref[...] = jnp.dot(x_ref[...], w_ref[0],
                             preferred_element_type=jnp.float32)

    return pl.pallas_call(
        body,
        grid=(nrb, _NC),
        in_specs=[
            pl.BlockSpec((_BLK, _FIN), lambda i, j: (i, 0)),
            pl.BlockSpec((1, _FIN, hw), lambda i, j: (j, 0, 0)),
        ],
        out_specs=pl.BlockSpec((_BLK, hw), lambda i, j: (j * nrb + i, 0)),
        out_shape=jax.ShapeDtypeStruct((_NC * _NPAD, hw), jnp.float32),
    )(xpad, w1s)


def _tc_scale1(xw2, degp):
    hw = _HID // _NC
    nrb = _NPAD // _BLK

    def body(xw_ref, deg_ref, o_ref):
        o_ref[...] = xw_ref[...] * _dd(deg_ref)

    return pl.pallas_call(
        body,
        grid=(nrb, _NC),
        in_specs=[
            pl.BlockSpec((_BLK, hw), lambda i, j: (j * nrb + i, 0)),
            pl.BlockSpec((_NC, _BLK, _LANES), lambda i, j: (0, i, 0)),
        ],
        out_specs=pl.BlockSpec((_BLK, hw), lambda i, j: (j * nrb + i, 0)),
        out_shape=jax.ShapeDtypeStruct((_NC * _NPAD, hw), jnp.float32),
    )(xw2, degp)


def _tc2(pp, table2, degp, w2p, b1r):
    hw = _HID // _NC
    nrb = _NPAD // _BLK

    def body(p_ref, xsl_ref, xsr_ref, deg_ref, w_ref, b_ref, o_ref):
        dd = _dd(deg_ref)
        xs = jnp.concatenate([xsl_ref[...], xsr_ref[...]], axis=1)
        h = jnp.maximum(dd * (p_ref[...] + xs) + b_ref[...], 0.0)
        o_ref[...] = jnp.dot(h, w_ref[...],
                             preferred_element_type=jnp.float32) * dd

    return pl.pallas_call(
        body,
        grid=(nrb,),
        in_specs=[
            pl.BlockSpec((_BLK, _HID), lambda i: (i, 0)),
            pl.BlockSpec((_BLK, hw), lambda i: (i, 0)),
            pl.BlockSpec((_BLK, hw), lambda i: (nrb + i, 0)),
            pl.BlockSpec((_NC, _BLK, _LANES), lambda i: (0, i, 0)),
            pl.BlockSpec((_HID, _CLSP), lambda i: (0, 0)),
            pl.BlockSpec((1, _HID), lambda i: (0, 0)),
        ],
        out_specs=pl.BlockSpec((_BLK, _CLSP), lambda i: (i, 0)),
        out_shape=jax.ShapeDtypeStruct((_NPAD, _CLSP), jnp.float32),
    )(pp, table2, table2, degp, w2p, b1r)


def _tc3(qp, hw_s, degp, b2r):
    def body(q_ref, hs_ref, deg_ref, b_ref, o_ref):
        logits = (_dd(deg_ref) * (q_ref[0] + q_ref[1] + hs_ref[...])
                  + b_ref[...])
        col = lax.broadcasted_iota(jnp.int32, (_BLK, _CLSP), 1)
        logits = jnp.where(col < _CLS, logits, -1e30)
        m = jnp.max(logits, axis=1, keepdims=True)
        lse = jnp.log(jnp.sum(jnp.exp(logits - m), axis=1, keepdims=True))
        o_ref[...] = logits - m - lse

    return pl.pallas_call(
        body,
        grid=(_NPAD // _BLK,),
        in_specs=[
            pl.BlockSpec((_NC, _BLK, _CLSP), lambda i: (0, i, 0)),
            pl.BlockSpec((_BLK, _CLSP), lambda i: (i, 0)),
            pl.BlockSpec((_NC, _BLK, _LANES), lambda i: (0, i, 0)),
            pl.BlockSpec((1, _CLSP), lambda i: (0, 0)),
        ],
        out_specs=pl.BlockSpec((_BLK, _CLSP), lambda i: (i, 0)),
        out_shape=jax.ShapeDtypeStruct((_NPAD, _CLSP), jnp.float32),
    )(qp, hw_s, degp, b2r)


def kernel(x, edge_index, W1, b1, W2, b2):
    e = edge_index.shape[1]
    ei = edge_index.astype(jnp.int32)

    per_round = _NBUF * _NW * _CHUNK
    epad = -(-e // per_round) * per_round
    ch = epad // (_NW * _CHUNK)
    ch2 = _NC * ch
    src = jnp.pad(ei[0], (0, epad - e), constant_values=_N)
    dst = jnp.pad(ei[1], (0, epad - e), constant_values=_N)
    src2d = src.reshape(_NS * ch2, _CHUNK)
    dst2d = dst.reshape(_NS * ch2, _CHUNK)
    srcoff2d = jnp.concatenate([src2d, src2d + _NPAD], axis=0)

    xpad = jnp.pad(x, ((0, _NPAD - _N), (0, 0)))
    w1s = W1.reshape(_FIN, _NC, _HID // _NC).transpose(1, 0, 2)
    w2p = jnp.pad(W2, ((0, 0), (0, _CLSP - _CLS)))
    b1r = b1.reshape(1, _HID)
    b2r = jnp.pad(b2, (0, _CLSP - _CLS)).reshape(1, _CLSP)

    ch1 = max(_NBUF, (2 * ch) * 3 // 20 // _NBUF * _NBUF)
    ch0 = 2 * ch - ch1

    xw2 = _tc_mm1(xpad, w1s)
    degp = _sc_degree(dst2d, ch)
    table2 = _tc_scale1(xw2, degp)
    pp = _sc_propagate_cols(table2, srcoff2d, dst2d, ch2)
    hw_s = _tc2(pp, table2, degp, w2p, b1r)
    qp = _sc_propagate(hw_s, src2d, dst2d, ch0, ch1, _CLSP)
    out = _tc3(qp, hw_s, degp, b2r)
    return out[:_N, :_CLS]


# speedup vs baseline: 1.1225x; 1.0099x over previous
import functools

import jax
import jax.numpy as jnp
from jax import lax
from jax.experimental import pallas as pl
from jax.experimental.pallas import tpu as pltpu
from jax.experimental.pallas import tpu_sc as plsc

_NC = 2
_NS = 16
_NW = _NC * _NS
_LANES = 16
_CHUNK = 128

_N = 10000
_NPAD = 10240
_RPT = _NPAD // _NS
_FIN = 128
_HID = 128
_CLS = 40
_CLSP = 48
_BLK = 512
_NBUF = 4


def _mesh():
    return plsc.VectorSubcoreMesh(
        core_axis_name="c", subcore_axis_name="s",
        num_cores=_NC, num_subcores=_NS)


_SC_PARAMS = pltpu.CompilerParams(use_tc_tiling_on_sc=False)


def _edge_pipeline(tab_hbm, acc, sidx, didx, rows, gsems, ssems, nch,
                   nbuf=_NBUF):
    depth = nbuf // 2
    for b in range(depth):
        pltpu.async_copy(tab_hbm.at[sidx.at[b]], rows.at[b], gsems[b])

    @pl.loop(0, nch, step=nbuf)
    def _(j):
        for b in range(nbuf):
            c = j + b
            b2 = (b + depth) % nbuf
            pltpu.make_async_copy(
                tab_hbm.at[sidx.at[c]], rows.at[b], gsems[b]).wait()
            pltpu.async_copy(rows.at[b], acc.at[didx.at[c]], ssems[b],
                             add=True)

            @pl.when(c + depth < nch)
            def _():
                @pl.when(c >= nbuf - depth)
                def _():
                    pltpu.make_async_copy(
                        rows.at[b2], acc.at[didx.at[c - (nbuf - depth)]],
                        ssems[b2]).wait()
                pltpu.async_copy(
                    tab_hbm.at[sidx.at[c + depth]], rows.at[b2], gsems[b2])

    for b in range(nbuf):
        pltpu.make_async_copy(
            rows.at[b], acc.at[didx.at[nch - nbuf + b]], ssems[b]).wait()


def _sc_degree(dst2d, ch):
    @functools.partial(
        pl.kernel,
        out_type=jax.ShapeDtypeStruct((_NC, _NPAD, _LANES), jnp.float32),
        mesh=_mesh(),
        compiler_params=_SC_PARAMS,
        scratch_types=[
            pltpu.VMEM((ch, _CHUNK), jnp.int32),
            pltpu.VMEM((_CHUNK, _LANES), jnp.float32),
            pltpu.VMEM((_RPT, _LANES), jnp.float32),
            pltpu.VMEM_SHARED((_NPAD, _LANES), jnp.float32),
        ],
    )
    def k(dst_hbm, out_hbm, idx_v, ones_v, zero_v, acc):
        cid = lax.axis_index("c")
        sid = lax.axis_index("s")
        wid = sid * _NC + cid

        @pl.loop(0, _CHUNK)
        def _(r):
            ones_v[r] = jnp.ones((_LANES,), jnp.float32)

        @pl.loop(0, _RPT)
        def _(r):
            zero_v[r] = jnp.zeros((_LANES,), jnp.float32)

        row0 = sid * _RPT
        pltpu.sync_copy(zero_v, acc.at[pl.ds(row0, _RPT)])
        pltpu.sync_copy(dst_hbm.at[pl.ds(wid * ch, ch)], idx_v)
        plsc.subcore_barrier()

        @pl.loop(0, ch)
        def _(j):
            pltpu.sync_copy(ones_v, acc.at[idx_v.at[j]], add=True)

        plsc.subcore_barrier()
        pltpu.sync_copy(acc.at[pl.ds(row0, _RPT)],
                        out_hbm.at[cid, pl.ds(row0, _RPT)])

    return k(dst2d)


def _sc_propagate_cols(table2, srcoff2d, dst2d, ch2):
    hw = _HID // _NC

    @functools.partial(
        pl.kernel,
        out_type=jax.ShapeDtypeStruct((_NPAD, _HID), jnp.float32),
        mesh=_mesh(),
        compiler_params=_SC_PARAMS,
        scratch_types=[
            pltpu.VMEM((ch2, _CHUNK), jnp.int32),
            pltpu.VMEM((ch2, _CHUNK), jnp.int32),
            pltpu.VMEM((_NBUF, _CHUNK, hw), jnp.float32),
            pltpu.VMEM_SHARED((_NPAD, hw), jnp.float32),
            [pltpu.SemaphoreType.DMA] * _NBUF,
            [pltpu.SemaphoreType.DMA] * _NBUF,
        ],
    )
    def k(tab_hbm, src_hbm, dst_hbm, out_hbm, sidx, didx, rows, acc,
          gsems, ssems):
        cid = lax.axis_index("c")
        sid = lax.axis_index("s")

        @pl.loop(0, _CHUNK)
        def _(r):
            @pl.loop(0, hw // _LANES)
            def _(q):
                rows[0, r, pl.ds(q * _LANES, _LANES)] = jnp.zeros(
                    (_LANES,), jnp.float32)

        row0 = sid * _RPT

        @pl.loop(0, _RPT // _CHUNK)
        def _(i):
            pltpu.sync_copy(rows.at[0],
                            acc.at[pl.ds(row0 + i * _CHUNK, _CHUNK)])

        pltpu.sync_copy(src_hbm.at[pl.ds((cid * _NS + sid) * ch2, ch2)], sidx)
        pltpu.sync_copy(dst_hbm.at[pl.ds(sid * ch2, ch2)], didx)
        plsc.subcore_barrier()
        _edge_pipeline(tab_hbm, acc, sidx, didx, rows, gsems, ssems,
                       ch2)
        plsc.subcore_barrier()
        pltpu.sync_copy(acc.at[pl.ds(row0, _RPT)],
                        out_hbm.at[pl.ds(row0, _RPT), pl.ds(cid * hw, hw)])

    return k(table2, srcoff2d, dst2d)


def _sc_propagate(table, src2d, dst2d, ch0, ch1, width, nbuf=4):
    chmax = max(ch0, ch1)
    assert ch0 % nbuf == 0 and ch1 % nbuf == 0

    @functools.partial(
        pl.kernel,
        out_type=jax.ShapeDtypeStruct((_NC, _NPAD, width), jnp.float32),
        mesh=_mesh(),
        compiler_params=_SC_PARAMS,
        scratch_types=[
            pltpu.VMEM((chmax, _CHUNK), jnp.int32),
            pltpu.VMEM((chmax, _CHUNK), jnp.int32),
            pltpu.VMEM((nbuf, _CHUNK, width), jnp.float32),
            pltpu.VMEM_SHARED((_NPAD, width), jnp.float32),
            [pltpu.SemaphoreType.DMA] * nbuf,
            [pltpu.SemaphoreType.DMA] * nbuf,
        ],
    )
    def k(tab_hbm, src_hbm, dst_hbm, out_hbm, sidx, didx, rows, acc,
          gsems, ssems):
        cid = lax.axis_index("c")
        sid = lax.axis_index("s")

        @pl.loop(0, _CHUNK)
        def _(r):
            @pl.loop(0, width // _LANES)
            def _(q):
                rows[0, r, pl.ds(q * _LANES, _LANES)] = jnp.zeros(
                    (_LANES,), jnp.float32)

        row0 = sid * _RPT

        @pl.loop(0, _RPT // _CHUNK)
        def _(i):
            pltpu.sync_copy(rows.at[0],
                            acc.at[pl.ds(row0 + i * _CHUNK, _CHUNK)])

        if ch0 > 0:
            @pl.when(cid == 0)
            def _():
                pltpu.sync_copy(src_hbm.at[pl.ds(sid * ch0, ch0)],
                                sidx.at[pl.ds(0, ch0)])
                pltpu.sync_copy(dst_hbm.at[pl.ds(sid * ch0, ch0)],
                                didx.at[pl.ds(0, ch0)])
                plsc.subcore_barrier()
                _edge_pipeline(tab_hbm, acc, sidx, didx, rows,
                               gsems, ssems, ch0, nbuf)

        if ch1 > 0:
            @pl.when(cid == 1)
            def _():
                pltpu.sync_copy(
                    src_hbm.at[pl.ds(_NS * ch0 + sid * ch1, ch1)],
                    sidx.at[pl.ds(0, ch1)])
                pltpu.sync_copy(
                    dst_hbm.at[pl.ds(_NS * ch0 + sid * ch1, ch1)],
                    didx.at[pl.ds(0, ch1)])
                plsc.subcore_barrier()
                _edge_pipeline(tab_hbm, acc, sidx, didx, rows,
                               gsems, ssems, ch1, nbuf)

        plsc.subcore_barrier()
        pltpu.sync_copy(acc.at[pl.ds(row0, _RPT)],
                        out_hbm.at[cid, pl.ds(row0, _RPT)])

    return k(table, src2d, dst2d)


def _dd(deg_ref):
    deg = deg_ref[0, :, 0:1] + deg_ref[1, :, 0:1] + 1.0
    return lax.rsqrt(deg)


def _tc_mm1(xpad, w1s):
    hw = _HID // _NC
    nrb = _NPAD // _BLK

    def body(x_ref, w_ref, o_ref):
        o_ref[...] = jnp.dot(x_ref[...], w_ref[0],
                             preferred_element_type=jnp.float32)

    return pl.pallas_call(
        body,
        grid=(nrb, _NC),
        in_specs=[
            pl.BlockSpec((_BLK, _FIN), lambda i, j: (i, 0)),
            pl.BlockSpec((1, _FIN, hw), lambda i, j: (j, 0, 0)),
        ],
        out_specs=pl.BlockSpec((_BLK, hw), lambda i, j: (j * nrb + i, 0)),
        out_shape=jax.ShapeDtypeStruct((_NC * _NPAD, hw), jnp.float32),
    )(xpad, w1s)


def _tc_scale1(xw2, degp):
    hw = _HID // _NC
    nrb = _NPAD // _BLK

    def body(xw_ref, deg_ref, o_ref):
        o_ref[...] = xw_ref[...] * _dd(deg_ref)

    return pl.pallas_call(
        body,
        grid=(nrb, _NC),
        in_specs=[
            pl.BlockSpec((_BLK, hw), lambda i, j: (j * nrb + i, 0)),
            pl.BlockSpec((_NC, _BLK, _LANES), lambda i, j: (0, i, 0)),
        ],
        out_specs=pl.BlockSpec((_BLK, hw), lambda i, j: (j * nrb + i, 0)),
        out_shape=jax.ShapeDtypeStruct((_NC * _NPAD, hw), jnp.float32),
    )(xw2, degp)


def _tc2(pp, table2, degp, w2p, b1r):
    hw = _HID // _NC
    nrb = _NPAD // _BLK

    def body(p_ref, xsl_ref, xsr_ref, deg_ref, w_ref, b_ref, o_ref):
        dd = _dd(deg_ref)
        xs = jnp.concatenate([xsl_ref[...], xsr_ref[...]], axis=1)
        h = jnp.maximum(dd * (p_ref[...] + xs) + b_ref[...], 0.0)
        o_ref[...] = jnp.dot(h, w_ref[...],
                             preferred_element_type=jnp.float32) * dd

    return pl.pallas_call(
        body,
        grid=(nrb,),
        in_specs=[
            pl.BlockSpec((_BLK, _HID), lambda i: (i, 0)),
            pl.BlockSpec((_BLK, hw), lambda i: (i, 0)),
            pl.BlockSpec((_BLK, hw), lambda i: (nrb + i, 0)),
            pl.BlockSpec((_NC, _BLK, _LANES), lambda i: (0, i, 0)),
            pl.BlockSpec((_HID, _CLSP), lambda i: (0, 0)),
            pl.BlockSpec((1, _HID), lambda i: (0, 0)),
        ],
        out_specs=pl.BlockSpec((_BLK, _CLSP), lambda i: (i, 0)),
        out_shape=jax.ShapeDtypeStruct((_NPAD, _CLSP), jnp.float32),
    )(pp, table2, table2, degp, w2p, b1r)


def _tc3(qp, hw_s, degp, b2r):
    def body(q_ref, hs_ref, deg_ref, b_ref, o_ref):
        logits = (_dd(deg_ref) * (q_ref[0] + q_ref[1] + hs_ref[...])
                  + b_ref[...])
        col = lax.broadcasted_iota(jnp.int32, (_BLK, _CLSP), 1)
        logits = jnp.where(col < _CLS, logits, -1e30)
        m = jnp.max(logits, axis=1, keepdims=True)
        lse = jnp.log(jnp.sum(jnp.exp(logits - m), axis=1, keepdims=True))
        o_ref[...] = logits - m - lse

    return pl.pallas_call(
        body,
        grid=(_NPAD // _BLK,),
        in_specs=[
            pl.BlockSpec((_NC, _BLK, _CLSP), lambda i: (0, i, 0)),
            pl.BlockSpec((_BLK, _CLSP), lambda i: (i, 0)),
            pl.BlockSpec((_NC, _BLK, _LANES), lambda i: (0, i, 0)),
            pl.BlockSpec((1, _CLSP), lambda i: (0, 0)),
        ],
        out_specs=pl.BlockSpec((_BLK, _CLSP), lambda i: (i, 0)),
        out_shape=jax.ShapeDtypeStruct((_NPAD, _CLSP), jnp.float32),
    )(qp, hw_s, degp, b2r)


def kernel(x, edge_index, W1, b1, W2, b2):
    e = edge_index.shape[1]
    ei = edge_index.astype(jnp.int32)

    per_round = _NBUF * _NW * _CHUNK
    epad = -(-e // per_round) * per_round
    ch = epad // (_NW * _CHUNK)
    ch2 = _NC * ch
    src = jnp.pad(ei[0], (0, epad - e), constant_values=_N)
    dst = jnp.pad(ei[1], (0, epad - e), constant_values=_N)
    src2d = src.reshape(_NS * ch2, _CHUNK)
    dst2d = dst.reshape(_NS * ch2, _CHUNK)
    srcoff2d = jnp.concatenate([src2d, src2d + _NPAD], axis=0)

    xpad = jnp.pad(x, ((0, _NPAD - _N), (0, 0)))
    w1s = W1.reshape(_FIN, _NC, _HID // _NC).transpose(1, 0, 2)
    w2p = jnp.pad(W2, ((0, 0), (0, _CLSP - _CLS)))
    b1r = b1.reshape(1, _HID)
    b2r = jnp.pad(b2, (0, _CLSP - _CLS)).reshape(1, _CLSP)

    ch1 = max(_NBUF, (2 * ch) * 2 // 20 // _NBUF * _NBUF)
    ch0 = 2 * ch - ch1

    xw2 = _tc_mm1(xpad, w1s)
    degp = _sc_degree(dst2d, ch)
    table2 = _tc_scale1(xw2, degp)
    pp = _sc_propagate_cols(table2, srcoff2d, dst2d, ch2)
    hw_s = _tc2(pp, table2, degp, w2p, b1r)
    qp = _sc_propagate(hw_s, src2d, dst2d, ch0, ch1, _CLSP)
    out = _tc3(qp, hw_s, degp, b2r)
    return out[:_N, :_CLS]
